# Initial kernel scaffold; baseline (speedup 1.0000x reference)
import jax
import jax.numpy as jnp
from jax.experimental import pallas as pl


def kernel(x, edge_index, Wl1, bl1, Wr1, Wl2, bl2, Wr2, Wfc, bfc):
    raise NotImplementedError("write your pallas kernel here")


# speedup vs baseline: 5.7915x; 5.7915x over previous
import functools

import jax
import jax.numpy as jnp
from jax import lax
from jax.experimental import pallas as pl
from jax.experimental.pallas import tpu as pltpu
from jax.experimental.pallas import tpu_sc as plsc

NUM_CORES = 2
NUM_SUBCORES = 16
NUM_WORKERS = NUM_CORES * NUM_SUBCORES
EDGE_BATCH = 80
LANES = 16


def _elu(v):
    return jnp.where(v > 0, v, jnp.exp(v) - 1.0)


def _dot_t(a, w):
    return lax.dot_general(a, w, dimension_numbers=(((1,), (1,)), ((), ())),
                           preferred_element_type=jnp.float32)


def _make_seg_sum(n_rows_tbl, width, n_pad, n_edges_total, with_deg):
    ept = n_edges_total // NUM_WORKERS
    assert ept * NUM_WORKERS == n_edges_total
    assert ept % EDGE_BATCH == 0
    nb = ept // EDGE_BATCH
    rpt = n_pad // NUM_SUBCORES
    assert rpt * NUM_SUBCORES == n_pad
    assert rpt % 128 == 0

    mesh = plsc.VectorSubcoreMesh(core_axis_name="c", subcore_axis_name="s")

    out_type = [jax.ShapeDtypeStruct((NUM_WORKERS, rpt, width), jnp.float32)]
    scratch = [
        pltpu.VMEM((EDGE_BATCH,), jnp.int32),
        pltpu.VMEM((EDGE_BATCH,), jnp.int32),
        pltpu.VMEM((EDGE_BATCH, width), jnp.float32),
        pltpu.VMEM_SHARED((n_pad, width), jnp.float32),
        pltpu.SemaphoreType.DMA,
    ]
    if with_deg:
        out_type.append(jax.ShapeDtypeStruct((NUM_WORKERS, rpt), jnp.float32))
        scratch += [
            pltpu.VMEM((EDGE_BATCH,), jnp.float32),
            pltpu.VMEM_SHARED((n_pad,), jnp.float32),
            pltpu.VMEM((rpt,), jnp.float32),
        ]

    @functools.partial(pl.kernel, mesh=mesh, out_type=out_type,
                       scratch_types=scratch)
    def seg(tbl, srcr, dstr, z, out, *rest):
        if with_deg:
            deg_out, src_v, dst_v, rows_v, acc, sem, ones_v, dacc, zrow_v = rest
        else:
            src_v, dst_v, rows_v, acc, sem = rest
        c = lax.axis_index("c")
        s = lax.axis_index("s")
        wid = c * NUM_SUBCORES + s

        pltpu.sync_copy(z.at[pl.ds(s * rpt, rpt)], acc.at[pl.ds(s * rpt, rpt)])
        if with_deg:
            for j in range(EDGE_BATCH // LANES):
                ones_v[pl.ds(j * LANES, LANES)] = jnp.ones((LANES,), jnp.float32)
            for j in range(rpt // LANES):
                zrow_v[pl.ds(j * LANES, LANES)] = jnp.zeros((LANES,), jnp.float32)
            pltpu.sync_copy(zrow_v, dacc.at[pl.ds(s * rpt, rpt)])
        plsc.subcore_barrier()

        ebase = wid * ept

        def body(i, carry):
            off = ebase + i * EDGE_BATCH
            pltpu.sync_copy(srcr.at[pl.ds(off, EDGE_BATCH)], src_v)
            pltpu.sync_copy(dstr.at[pl.ds(off, EDGE_BATCH)], dst_v)
            pltpu.async_copy(tbl.at[src_v], rows_v, sem).wait()
            pltpu.sync_copy(rows_v, acc.at[dst_v], add=True)
            if with_deg:
                pltpu.sync_copy(ones_v, dacc.at[dst_v], add=True)
            return carry

        lax.fori_loop(0, nb, body, 0)

        plsc.subcore_barrier()
        pltpu.sync_copy(acc.at[pl.ds(s * rpt, rpt)], out.at[wid])
        if with_deg:
            pltpu.sync_copy(dacc.at[pl.ds(s * rpt, rpt)], deg_out.at[wid])

    return seg


def _dense1_body(x_ref, p0_ref, p1_ref, d0_ref, d1_ref, wl1_ref, bl1_ref,
                 wr1_ref, wl2_ref, wr2_ref, bl2_ref, g_ref, r_ref, dinv_ref):
    deg = d0_ref[...] + d1_ref[...]
    dinv = 1.0 / jnp.maximum(deg, 1.0)
    agg = (p0_ref[...] + p1_ref[...]) * dinv
    out = _dot_t(agg, wl1_ref[...]) + bl1_ref[...] + _dot_t(x_ref[...], wr1_ref[...])
    nrm = jnp.sqrt(jnp.sum(out * out, axis=1, keepdims=True))
    h = _elu(out / jnp.maximum(nrm, 1e-12))
    g = _dot_t(h, wl2_ref[...])
    r = _dot_t(h, wr2_ref[...]) + bl2_ref[...]
    g_ref[...] = jnp.stack([g[:, :128], g[:, 128:]], axis=0)
    r_ref[...] = r
    dinv_ref[...] = dinv


def _dense2_body(a0_ref, a1_ref, dinv_ref, r_ref, wfc_ref, bfc_ref,
                 y_ref, acc_ref):
    i = pl.program_id(0)
    n_i = pl.num_programs(0)
    a = jnp.concatenate([a0_ref[...], a1_ref[...]], axis=1)
    out = a * dinv_ref[...] + r_ref[...]
    nrm = jnp.sqrt(jnp.sum(out * out, axis=1, keepdims=True))
    h2 = _elu(out / jnp.maximum(nrm, 1e-12))
    bm = jnp.max(h2, axis=0, keepdims=True)

    @pl.when(i == 0)
    def _():
        acc_ref[...] = jnp.broadcast_to(bm, acc_ref.shape)

    @pl.when(i > 0)
    def _():
        acc_ref[...] = jnp.maximum(acc_ref[...], bm)

    @pl.when(i == n_i - 1)
    def _():
        m = _elu(acc_ref[0:1, :])
        y_ref[...] = _dot_t(m, wfc_ref[...]) + bfc_ref[...]


def kernel(x, edge_index, Wl1, bl1, Wr1, Wl2, bl2, Wr2, Wfc, bfc):
    n, d = x.shape
    e = edge_index.shape[1]
    h1 = Wl1.shape[0]
    h2 = Wl2.shape[0]

    src = edge_index[0].astype(jnp.int32)
    dst = edge_index[1].astype(jnp.int32)

    n_pad = ((n + 2047) // 2048) * 2048

    z1 = jnp.zeros((n_pad, d), jnp.float32)
    p, dp = _make_seg_sum(n, d, n_pad, e, True)(x, src, dst, z1)
    p = p.reshape(NUM_CORES, n_pad, d)[:, :n]
    dd = dp.reshape(NUM_CORES, n_pad)[:, :n]
    d0 = dd[0].reshape(n, 1)
    d1 = dd[1].reshape(n, 1)

    bn = 1000
    grid1 = (n // bn,)
    g, r, dinv = pl.pallas_call(
        _dense1_body,
        grid=grid1,
        in_specs=[
            pl.BlockSpec((bn, d), lambda i: (i, 0)),
            pl.BlockSpec((bn, d), lambda i: (i, 0)),
            pl.BlockSpec((bn, d), lambda i: (i, 0)),
            pl.BlockSpec((bn, 1), lambda i: (i, 0)),
            pl.BlockSpec((bn, 1), lambda i: (i, 0)),
            pl.BlockSpec((h1, d), lambda i: (0, 0)),
            pl.BlockSpec((1, h1), lambda i: (0, 0)),
            pl.BlockSpec((h1, d), lambda i: (0, 0)),
            pl.BlockSpec((h2, h1), lambda i: (0, 0)),
            pl.BlockSpec((h2, h1), lambda i: (0, 0)),
            pl.BlockSpec((1, h2), lambda i: (0, 0)),
        ],
        out_specs=[
            pl.BlockSpec((2, bn, 128), lambda i: (0, i, 0)),
            pl.BlockSpec((bn, h2), lambda i: (i, 0)),
            pl.BlockSpec((bn, 1), lambda i: (i, 0)),
        ],
        out_shape=[
            jax.ShapeDtypeStruct((2, n, 128), jnp.float32),
            jax.ShapeDtypeStruct((n, h2), jnp.float32),
            jax.ShapeDtypeStruct((n, 1), jnp.float32),
        ],
    )(x, p[0], p[1], d0, d1, Wl1, bl1.reshape(1, h1), Wr1, Wl2, Wr2,
      bl2.reshape(1, h2))

    gt = g.reshape(2 * n, 128)
    src2 = jnp.concatenate([src, src + n])
    dst2 = jnp.concatenate([dst, dst])
    z2 = jnp.zeros((n_pad, 128), jnp.float32)
    (a,) = _make_seg_sum(2 * n, 128, n_pad, 2 * e, False)(gt, src2, dst2, z2)
    a = a.reshape(NUM_CORES, n_pad, 128)[:, :n]

    y = pl.pallas_call(
        _dense2_body,
        grid=grid1,
        in_specs=[
            pl.BlockSpec((bn, 128), lambda i: (i, 0)),
            pl.BlockSpec((bn, 128), lambda i: (i, 0)),
            pl.BlockSpec((bn, 1), lambda i: (i, 0)),
            pl.BlockSpec((bn, h2), lambda i: (i, 0)),
            pl.BlockSpec((d, h2), lambda i: (0, 0)),
            pl.BlockSpec((1, d), lambda i: (0, 0)),
        ],
        out_specs=pl.BlockSpec((1, d), lambda i: (0, 0)),
        out_shape=jax.ShapeDtypeStruct((1, d), jnp.float32),
        scratch_shapes=[pltpu.VMEM((8, h2), jnp.float32)],
    )(a[0], a[1], dinv, r, Wfc, bfc.reshape(1, d))

    return y.reshape(-1)
